# baseline (speedup 1.0000x reference)
import jax
import jax.numpy as jnp
from jax.experimental import pallas as pl
from jax.experimental.pallas import tpu as pltpu

E = 160000
D = 128
H = 256
ORB = 13
OO = ORB * ORB
NUM_EXPERTS = 4
BOUNDS = (1.2, 1.6, 2.0)

TM = 8000


def _fused_body(vec_ref, feat_ref, w1_ref, b1_ref, w2_ref, b2_ref, out_ref):
    vec = vec_ref[...]
    feat = feat_ref[...].astype(jnp.bfloat16)
    d2 = jnp.sum(vec * vec, axis=1)

    res = None
    for i in range(NUM_EXPERTS):
        h = jnp.maximum(
            jnp.dot(feat, w1_ref[i], preferred_element_type=jnp.float32)
            + b1_ref[i][None, :], 0.0).astype(jnp.bfloat16)
        o = (jnp.dot(h, w2_ref[i], preferred_element_type=jnp.float32)
             + b2_ref[i][None, :])
        if i == 0:
            res = o
        else:
            lo = BOUNDS[i - 1] * BOUNDS[i - 1]
            if i < NUM_EXPERTS - 1:
                hi = BOUNDS[i] * BOUNDS[i]
                m = (d2 >= lo) & (d2 < hi)
            else:
                m = d2 >= lo
            res = jnp.where(m[:, None], o, res)
    out_ref[...] = res.astype(jnp.bfloat16)


def kernel(edge_vec, edge_feat, W1, b1, W2, b2):
    grid = E // TM
    out = pl.pallas_call(
        _fused_body,
        grid=(grid,),
        in_specs=[
            pl.BlockSpec((TM, 3), lambda i: (i, 0)),
            pl.BlockSpec((TM, D), lambda i: (i, 0)),
            pl.BlockSpec((NUM_EXPERTS, D, H), lambda i: (0, 0, 0)),
            pl.BlockSpec((NUM_EXPERTS, H), lambda i: (0, 0)),
            pl.BlockSpec((NUM_EXPERTS, H, OO), lambda i: (0, 0, 0)),
            pl.BlockSpec((NUM_EXPERTS, OO), lambda i: (0, 0)),
        ],
        out_specs=pl.BlockSpec((TM, OO), lambda i: (i, 0)),
        out_shape=jax.ShapeDtypeStruct((E, OO), jnp.bfloat16),
        compiler_params=pltpu.CompilerParams(
            dimension_semantics=("parallel",),
        ),
    )(edge_vec, edge_feat,
      W1.astype(jnp.bfloat16), b1, W2.astype(jnp.bfloat16), b2)
    return out
